# Initial kernel scaffold; baseline (speedup 1.0000x reference)
import jax
import jax.numpy as jnp
from jax.experimental import pallas as pl


def kernel(spacing_bins, weights, nears, fars):
    raise NotImplementedError("write your pallas kernel here")


# speedup vs baseline: 3.4480x; 3.4480x over previous
import functools

import jax
import jax.numpy as jnp
import numpy as np
from jax import lax
from jax.experimental import pallas as pl
from jax.experimental.pallas import tpu as pltpu
from jax.experimental.pallas import tpu_sc as plsc

R = 65536
S = 64
NB = S + 1
OUTW = 2 * S + 1
L = 16
HIST_PAD = 0.01
EPS = 1e-5

NC, NS = 2, 16
NW = NC * NS
GROUPS = R // L
GPW = GROUPS // NW

_f32 = jnp.float32
_i32 = jnp.int32


def _body(sp_hbm, wt_hbm, nr_hbm, fr_hbm, tt_hbm, out_hbm,
          tt_v, sp_v, wt_v, nr_v, fr_v, cdf_v, hi_v, hb_v, out_v):
    wid = lax.axis_index("c") * NS + lax.axis_index("s")
    lane = lax.iota(_i32, L)

    pltpu.sync_copy(tt_hbm, tt_v)

    zeros_f = jnp.zeros((L,), _f32)
    zeros_i = jnp.zeros((L,), _i32)
    ones_i = jnp.ones((L,), _i32)

    for j in range(NB):
        hi_v[pl.ds(j * L, L)] = zeros_i
    for i in range(S):
        hb_v[pl.ds(i * L, L)] = zeros_i

    def group(g, carry):
        base = (wid * GPW + g)
        pltpu.sync_copy(sp_hbm.at[pl.ds(base * (L * NB), L * NB)], sp_v)
        pltpu.sync_copy(wt_hbm.at[pl.ds(base * (L * S), L * S)], wt_v)
        pltpu.sync_copy(nr_hbm.at[pl.ds(base * L, L)], nr_v)
        pltpu.sync_copy(fr_hbm.at[pl.ds(base * L, L)], fr_v)

        near = nr_v[...]
        scale = fr_v[...] - near
        wlane = lane * S

        acc = zeros_f
        for s in range(S):
            acc = acc + plsc.load_gather(wt_v, [wlane + s])
        wsum = acc + jnp.full((L,), _f32(S) * _f32(HIST_PAD), _f32)
        padding = jnp.maximum(jnp.full((L,), _f32(EPS), _f32) - wsum, zeros_f)
        wsum = wsum + padding
        winc = padding * _f32(1.0 / S) + jnp.full((L,), _f32(HIST_PAD), _f32)
        rw = jnp.full((L,), _f32(1.0), _f32) / wsum

        hi_v[pl.ds(0, L)] = ones_i
        cdf_v[pl.ds(0, L)] = zeros_f
        one_f = jnp.full((L,), _f32(1.0), _f32)
        accp = zeros_f
        for s in range(S):
            w = plsc.load_gather(wt_v, [wlane + s])
            accp = accp + (w + winc) * rw
            c = jnp.minimum(one_f, accp)
            cdf_v[pl.ds((s + 1) * L, L)] = c
            y = jnp.clip(jnp.full((L,), _f32(NB), _f32) * c - _f32(0.5),
                         0.0, _f32(NB))
            a = y.astype(_i32)
            a = jnp.where(a.astype(_f32) < y, a + 1, a)
            g0 = plsc.load_gather(tt_v, [a])
            g1 = plsc.load_gather(tt_v, [a + 1])
            a = jnp.where(g0 >= c, a - 1, a)
            a = jnp.where(g1 < c, a + 1, a)
            plsc.addupdate_scatter(hi_v, [a * L + lane], ones_i)

        splane = lane * NB
        olane = lane * OUTW
        indc = zeros_i
        s64 = jnp.full((L,), _i32(S), _i32)
        for j in range(NB):
            h = hi_v[pl.ds(j * L, L)]
            hi_v[pl.ds(j * L, L)] = zeros_i
            indc = indc + h
            below = jnp.minimum(indc - 1, s64)
            above = jnp.minimum(indc, s64)
            cdf0 = plsc.load_gather(cdf_v, [below * L + lane])
            cdf1 = plsc.load_gather(cdf_v, [above * L + lane])
            b0 = plsc.load_gather(sp_v, [splane + below])
            b1 = plsc.load_gather(sp_v, [splane + above])
            uj = plsc.load_gather(tt_v, [jnp.full((L,), _i32(j + 1), _i32)])
            q = (uj - cdf0) / (cdf1 - cdf0)
            q = jnp.where(q != q, zeros_f, q)
            t = jnp.clip(q, 0.0, 1.0)
            bnew = b0 + t * (b1 - b0)
            if j < S:
                val = near + bnew * scale
                posb = jnp.minimum(below + 1, s64) + _i32(j)
                plsc.store_scatter(out_v, [olane + posb], val)
                plsc.addupdate_scatter(hb_v, [below * L + lane], ones_i)
            else:
                spend = plsc.load_gather(sp_v, [splane + S])
                val = near + jnp.maximum(spend, bnew) * scale
                plsc.store_scatter(out_v, [olane + _i32(OUTW - 1)], val)

        carrya = zeros_i
        for i in range(S):
            posa = carrya + _i32(i)
            spi = plsc.load_gather(sp_v, [splane + i])
            val = near + spi * scale
            plsc.store_scatter(out_v, [olane + posa], val)
            h = hb_v[pl.ds(i * L, L)]
            hb_v[pl.ds(i * L, L)] = zeros_i
            carrya = carrya + h

        pltpu.sync_copy(out_v, out_hbm.at[pl.ds(base * (L * OUTW), L * OUTW)])
        return carry

    lax.fori_loop(0, GPW, group, 0)


@functools.partial(jax.jit)
def kernel(spacing_bins, weights, nears, fars):
    u = jnp.linspace(0.0, 1.0 - 1.0 / NB, NB, dtype=_f32) + _f32(1.0 / (2 * NB))
    tt = jnp.concatenate([
        jnp.full((1,), -np.inf, _f32), u, jnp.full((1,), np.inf, _f32),
        jnp.zeros((80 - NB - 2,), _f32)])

    mesh = plsc.VectorSubcoreMesh(core_axis_name="c", subcore_axis_name="s")
    k = pl.kernel(
        _body,
        out_type=jax.ShapeDtypeStruct((R * OUTW,), _f32),
        mesh=mesh,
        compiler_params=pltpu.CompilerParams(needs_layout_passes=False),
        scratch_types=[
            pltpu.VMEM((80,), _f32),
            pltpu.VMEM((L * NB,), _f32),
            pltpu.VMEM((L * S,), _f32),
            pltpu.VMEM((L,), _f32),
            pltpu.VMEM((L,), _f32),
            pltpu.VMEM((NB * L,), _f32),
            pltpu.VMEM(((NB + 1) * L,), _i32),
            pltpu.VMEM(((S + 1) * L,), _i32),
            pltpu.VMEM((L * OUTW,), _f32),
        ],
    )
    out = k(spacing_bins.reshape(-1), weights.reshape(-1),
            nears.reshape(-1), fars.reshape(-1), tt)
    return out.reshape(R, OUTW)


# speedup vs baseline: 4.8610x; 1.4098x over previous
import functools

import jax
import jax.numpy as jnp
import numpy as np
from jax import lax
from jax.experimental import pallas as pl
from jax.experimental.pallas import tpu as pltpu
from jax.experimental.pallas import tpu_sc as plsc

R = 65536
S = 64
NB = S + 1
OUTW = 2 * S + 1
L = 16
HIST_PAD = 0.01
EPS = 1e-5

NC, NS = 2, 16
NW = NC * NS
GROUPS = R // L
GPW = GROUPS // NW

SPW = L * NB
WTW = L * S
NFW = 2 * L
OUW = L * OUTW

_f32 = jnp.float32
_i32 = jnp.int32

_UJ = (np.linspace(0.0, 1.0 - 1.0 / NB, NB, dtype=np.float32)
       + np.float32(1.0 / (2 * NB)))


def _body(sp_hbm, wt_hbm, nf_hbm, out_hbm,
          sp_v, wt_v, nf_v, out_v, cdf_v, hi_v, hb_v,
          s_in0, s_in1, s_out0, s_out1):
    wid = lax.axis_index("c") * NS + lax.axis_index("s")
    lane = lax.iota(_i32, L)
    wlane = lane * S
    splane = lane * NB
    olane = lane * OUTW
    gbase = wid * GPW

    zeros_f = jnp.zeros((L,), _f32)
    zeros_i = jnp.zeros((L,), _i32)
    ones_i = jnp.ones((L,), _i32)
    one_f = jnp.full((L,), _f32(1.0), _f32)
    s64 = jnp.full((L,), _i32(S), _i32)

    for j in range(NB):
        hi_v[pl.ds(j * L, L)] = zeros_i
    for i in range(S):
        hb_v[pl.ds(i * L, L)] = zeros_i
    cdf_v[pl.ds(0, L)] = zeros_f

    def _in_copies(g, half, sem):
        base = gbase + g
        return (
            (sp_hbm.at[pl.ds(base * SPW, SPW)],
             sp_v.at[pl.ds(half * SPW, SPW)], sem),
            (wt_hbm.at[pl.ds(base * WTW, WTW)],
             wt_v.at[pl.ds(half * WTW, WTW)], sem),
            (nf_hbm.at[pl.ds(base * NFW, NFW)],
             nf_v.at[pl.ds(half * NFW, NFW)], sem),
        )

    def issue_in(g, half, sem):
        for src, dst, sm in _in_copies(g, half, sem):
            pltpu.async_copy(src, dst, sm)

    def wait_in(g, half, sem):
        for src, dst, sm in _in_copies(g, half, sem):
            pltpu.make_async_copy(src, dst, sm).wait()

    issue_in(0, 0, s_in0)

    def group(g, carry):
        b = jnp.bitwise_and(g, 1)
        is0 = b == 0
        gnext = jnp.minimum(g + 1, GPW - 1)

        @pl.when(is0)
        def _():
            wait_in(g, 0, s_in0)
            issue_in(gnext, 1, s_in1)

        @pl.when(jnp.logical_not(is0))
        def _():
            wait_in(g, 1, s_in1)
            issue_in(gnext, 0, s_in0)

        @pl.when(jnp.logical_and(g >= 2, is0))
        def _():
            pltpu.make_async_copy(
                out_v.at[pl.ds(0, OUW)],
                out_hbm.at[pl.ds((gbase + g - 2) * OUW, OUW)],
                s_out0).wait()

        @pl.when(jnp.logical_and(g >= 2, jnp.logical_not(is0)))
        def _():
            pltpu.make_async_copy(
                out_v.at[pl.ds(OUW, OUW)],
                out_hbm.at[pl.ds((gbase + g - 2) * OUW, OUW)],
                s_out1).wait()

        wlane_b = wlane + b * WTW
        splane_b = splane + b * SPW
        olane_b = olane + b * OUW
        nfoff = b * NFW
        near = plsc.load_gather(nf_v, [lane * 2 + nfoff])
        far = plsc.load_gather(nf_v, [lane * 2 + (nfoff + 1)])
        scale = far - near

        acc = zeros_f
        for s in range(S):
            w = plsc.load_gather(wt_v, [wlane_b + s])
            acc = acc + w
            cdf_v[pl.ds((s + 1) * L, L)] = acc

        wsum = acc + jnp.full((L,), _f32(S) * _f32(HIST_PAD), _f32)
        padding = jnp.maximum(jnp.full((L,), _f32(EPS), _f32) - wsum, zeros_f)
        wsum = wsum + padding
        rw = one_f / wsum
        hpr = (jnp.full((L,), _f32(HIST_PAD), _f32)
               + padding * _f32(1.0 / S)) * rw

        hi_v[pl.ds(0, L)] = ones_i
        kacc = zeros_f
        for s in range(S):
            a = cdf_v[pl.ds((s + 1) * L, L)]
            kacc = kacc + hpr
            c = jnp.minimum(one_f, a * rw + kacc)
            cdf_v[pl.ds((s + 1) * L, L)] = c
            y = c * _f32(NB) - _f32(0.5)
            ai = y.astype(_i32)
            ai = jnp.where(ai.astype(_f32) < y, ai + 1, ai)
            plsc.addupdate_scatter(hi_v, [ai * L + lane], ones_i)

        indc = zeros_i
        for j in range(NB):
            h = hi_v[pl.ds(j * L, L)]
            hi_v[pl.ds(j * L, L)] = zeros_i
            indc = indc + h
            below = indc - 1
            above = jnp.minimum(indc, s64)
            ib = below * L + lane
            ia = above * L + lane
            cdf0 = plsc.load_gather(cdf_v, [ib])
            cdf1 = plsc.load_gather(cdf_v, [ia])
            b0 = plsc.load_gather(sp_v, [splane_b + below])
            b1 = plsc.load_gather(sp_v, [splane_b + above])
            uj = jnp.full((L,), _f32(_UJ[j]), _f32)
            q = (uj - cdf0) / (cdf1 - cdf0)
            q = jnp.where(q != q, zeros_f, q)
            t = jnp.clip(q, 0.0, 1.0)
            bnew = b0 + t * (b1 - b0)
            if j < S:
                val = near + bnew * scale
                posb = above + _i32(j)
                plsc.store_scatter(out_v, [olane_b + posb], val)
                plsc.addupdate_scatter(hb_v, [ib], ones_i)
            else:
                spend = plsc.load_gather(sp_v, [splane_b + S])
                val = near + jnp.maximum(spend, bnew) * scale
                plsc.store_scatter(out_v, [olane_b + _i32(OUTW - 1)], val)

        carrya = zeros_i
        for i in range(S):
            posa = carrya + _i32(i)
            spi = plsc.load_gather(sp_v, [splane_b + i])
            val = near + spi * scale
            plsc.store_scatter(out_v, [olane_b + posa], val)
            h = hb_v[pl.ds(i * L, L)]
            hb_v[pl.ds(i * L, L)] = zeros_i
            carrya = carrya + h

        @pl.when(is0)
        def _():
            pltpu.async_copy(out_v.at[pl.ds(0, OUW)],
                             out_hbm.at[pl.ds((gbase + g) * OUW, OUW)],
                             s_out0)

        @pl.when(jnp.logical_not(is0))
        def _():
            pltpu.async_copy(out_v.at[pl.ds(OUW, OUW)],
                             out_hbm.at[pl.ds((gbase + g) * OUW, OUW)],
                             s_out1)

        return carry

    lax.fori_loop(0, GPW, group, 0)

    pltpu.make_async_copy(
        out_v.at[pl.ds(0, OUW)],
        out_hbm.at[pl.ds((gbase + GPW - 2) * OUW, OUW)], s_out0).wait()
    pltpu.make_async_copy(
        out_v.at[pl.ds(OUW, OUW)],
        out_hbm.at[pl.ds((gbase + GPW - 1) * OUW, OUW)], s_out1).wait()
    wait_in(GPW - 1, 0, s_in0)


@functools.partial(jax.jit)
def kernel(spacing_bins, weights, nears, fars):
    nf = jnp.concatenate([nears, fars], axis=1)

    mesh = plsc.VectorSubcoreMesh(core_axis_name="c", subcore_axis_name="s")
    k = pl.kernel(
        _body,
        out_type=jax.ShapeDtypeStruct((R * OUTW,), _f32),
        mesh=mesh,
        compiler_params=pltpu.CompilerParams(needs_layout_passes=False),
        scratch_types=[
            pltpu.VMEM((2 * SPW,), _f32),
            pltpu.VMEM((2 * WTW,), _f32),
            pltpu.VMEM((2 * NFW,), _f32),
            pltpu.VMEM((2 * OUW,), _f32),
            pltpu.VMEM((NB * L,), _f32),
            pltpu.VMEM(((NB + 1) * L,), _i32),
            pltpu.VMEM((NB * L,), _i32),
            pltpu.SemaphoreType.DMA,
            pltpu.SemaphoreType.DMA,
            pltpu.SemaphoreType.DMA,
            pltpu.SemaphoreType.DMA,
        ],
    )
    out = k(spacing_bins.reshape(-1), weights.reshape(-1),
            nf.reshape(-1))
    return out.reshape(R, OUTW)


# speedup vs baseline: 11.6855x; 2.4039x over previous
import functools

import jax
import jax.numpy as jnp
import numpy as np
from jax import lax
from jax.experimental import pallas as pl
from jax.experimental.pallas import tpu as pltpu
from jax.experimental.pallas import tpu_sc as plsc

R = 65536
S = 64
NB = S + 1
OUTW = 2 * S + 1
L = 16
HIST_PAD = 0.01
EPS = 1e-5

NC, NS = 2, 16
NW = NC * NS
GROUPS = R // L
GPW = GROUPS // NW

SPW = L * NB
WTW = L * S
NFW = 2 * L
OUW = L * OUTW

_f32 = jnp.float32
_i32 = jnp.int32

_UJ = (np.linspace(0.0, 1.0 - 1.0 / NB, NB, dtype=np.float32)
       + np.float32(1.0 / (2 * NB)))
_USTEP = float(np.float32(1.0 / NB))
_UHALF = float(np.float32(1.0 / (2 * NB)))


def _body(sp_hbm, wt_hbm, nf_hbm, out_hbm,
          sp_v, wt_v, nf_v, out_v, cdf_v, hi_v, hb_v,
          s_in0, s_in1, s_out0, s_out1):
    wid = lax.axis_index("c") * NS + lax.axis_index("s")
    lane = lax.iota(_i32, L)
    wlane = lane * S
    splane = lane * NB
    olane = lane * OUTW
    gbase = wid * GPW

    zeros_f = jnp.zeros((L,), _f32)
    zeros_i = jnp.zeros((L,), _i32)
    ones_i = jnp.ones((L,), _i32)
    one_f = jnp.full((L,), _f32(1.0), _f32)
    s64 = jnp.full((L,), _i32(S), _i32)

    for j in range(NB):
        hi_v[pl.ds(j * L, L)] = zeros_i
    for i in range(S):
        hb_v[pl.ds(i * L, L)] = zeros_i
    cdf_v[pl.ds(0, L)] = zeros_f

    def _in_copies(g, half, sem):
        base = gbase + g
        return (
            (sp_hbm.at[pl.ds(base * SPW, SPW)],
             sp_v.at[pl.ds(half * SPW, SPW)], sem),
            (wt_hbm.at[pl.ds(base * WTW, WTW)],
             wt_v.at[pl.ds(half * WTW, WTW)], sem),
            (nf_hbm.at[pl.ds(base * NFW, NFW)],
             nf_v.at[pl.ds(half * NFW, NFW)], sem),
        )

    def issue_in(g, half, sem):
        for src, dst, sm in _in_copies(g, half, sem):
            pltpu.async_copy(src, dst, sm)

    def wait_in(g, half, sem):
        for src, dst, sm in _in_copies(g, half, sem):
            pltpu.make_async_copy(src, dst, sm).wait()

    issue_in(0, 0, s_in0)

    def group(g, carry):
        b = jnp.bitwise_and(g, 1)
        is0 = b == 0
        gnext = jnp.minimum(g + 1, GPW - 1)

        @pl.when(is0)
        def _():
            wait_in(g, 0, s_in0)
            issue_in(gnext, 1, s_in1)

        @pl.when(jnp.logical_not(is0))
        def _():
            wait_in(g, 1, s_in1)
            issue_in(gnext, 0, s_in0)

        @pl.when(jnp.logical_and(g >= 2, is0))
        def _():
            pltpu.make_async_copy(
                out_v.at[pl.ds(0, OUW)],
                out_hbm.at[pl.ds((gbase + g - 2) * OUW, OUW)],
                s_out0).wait()

        @pl.when(jnp.logical_and(g >= 2, jnp.logical_not(is0)))
        def _():
            pltpu.make_async_copy(
                out_v.at[pl.ds(OUW, OUW)],
                out_hbm.at[pl.ds((gbase + g - 2) * OUW, OUW)],
                s_out1).wait()

        wlane_b = wlane + b * WTW
        splane_b = splane + b * SPW
        olane_b = olane + b * OUW
        nfoff = b * NFW
        near = plsc.load_gather(nf_v, [lane * 2 + nfoff])
        far = plsc.load_gather(nf_v, [lane * 2 + (nfoff + 1)])
        scale = far - near

        @plsc.parallel_loop(0, S, unroll=8, carry=zeros_f)
        def _p1(s, acc):
            w = plsc.load_gather(wt_v, [wlane_b + s])
            acc = acc + w
            cdf_v[pl.ds(s * L + L, L)] = acc
            return acc

        acc = _p1
        wsum = acc + jnp.full((L,), _f32(S) * _f32(HIST_PAD), _f32)
        padding = jnp.maximum(jnp.full((L,), _f32(EPS), _f32) - wsum, zeros_f)
        wsum = wsum + padding
        rw = one_f / wsum
        hpr = (jnp.full((L,), _f32(HIST_PAD), _f32)
               + padding * _f32(1.0 / S)) * rw

        hi_v[pl.ds(0, L)] = ones_i

        @plsc.parallel_loop(0, S, unroll=8)
        def _p2(s):
            a = cdf_v[pl.ds(s * L + L, L)]
            sf = (s + 1).astype(_f32)
            c = jnp.minimum(one_f, a * rw + hpr * sf)
            cdf_v[pl.ds(s * L + L, L)] = c
            y = c * _f32(NB) + _f32(0.5)
            ai = y.astype(_i32)
            plsc.addupdate_scatter(hi_v, [ai * L + lane], ones_i)

        @plsc.parallel_loop(0, S, unroll=4, carry=zeros_i)
        def _p3(j, indc):
            h = hi_v[pl.ds(j * L, L)]
            hi_v[pl.ds(j * L, L)] = zeros_i
            indc = indc + h
            below = indc - 1
            above = jnp.minimum(indc, s64)
            ib = below * L + lane
            ia = above * L + lane
            cdf0 = plsc.load_gather(cdf_v, [ib])
            cdf1 = plsc.load_gather(cdf_v, [ia])
            b0 = plsc.load_gather(sp_v, [splane_b + below])
            b1 = plsc.load_gather(sp_v, [splane_b + above])
            uj = j.astype(_f32) * _f32(_USTEP) + jnp.full((L,), _f32(_UHALF),
                                                          _f32)
            q = (uj - cdf0) / jnp.maximum(cdf1 - cdf0, _f32(1e-37))
            t = jnp.clip(q, 0.0, 1.0)
            bnew = b0 + t * (b1 - b0)
            val = near + bnew * scale
            posb = above + j
            plsc.store_scatter(out_v, [olane_b + posb], val)
            plsc.addupdate_scatter(hb_v, [ib], ones_i)
            return indc

        indc = _p3 + hi_v[pl.ds(S * L, L)]
        hi_v[pl.ds(S * L, L)] = zeros_i
        below = indc - 1
        above = jnp.minimum(indc, s64)
        cdf0 = plsc.load_gather(cdf_v, [below * L + lane])
        cdf1 = plsc.load_gather(cdf_v, [above * L + lane])
        b0 = plsc.load_gather(sp_v, [splane_b + below])
        b1 = plsc.load_gather(sp_v, [splane_b + above])
        uj = jnp.full((L,), _f32(_UJ[S]), _f32)
        q = (uj - cdf0) / jnp.maximum(cdf1 - cdf0, _f32(1e-37))
        t = jnp.clip(q, 0.0, 1.0)
        bnew = b0 + t * (b1 - b0)
        spend = plsc.load_gather(sp_v, [splane_b + S])
        val = near + jnp.maximum(spend, bnew) * scale
        plsc.store_scatter(out_v, [olane_b + _i32(OUTW - 1)], val)

        @plsc.parallel_loop(0, S, unroll=8, carry=zeros_i)
        def _p4(i, carrya):
            posa = carrya + i
            spi = plsc.load_gather(sp_v, [splane_b + i])
            val = near + spi * scale
            plsc.store_scatter(out_v, [olane_b + posa], val)
            h = hb_v[pl.ds(i * L, L)]
            hb_v[pl.ds(i * L, L)] = zeros_i
            return carrya + h

        @pl.when(is0)
        def _():
            pltpu.async_copy(out_v.at[pl.ds(0, OUW)],
                             out_hbm.at[pl.ds((gbase + g) * OUW, OUW)],
                             s_out0)

        @pl.when(jnp.logical_not(is0))
        def _():
            pltpu.async_copy(out_v.at[pl.ds(OUW, OUW)],
                             out_hbm.at[pl.ds((gbase + g) * OUW, OUW)],
                             s_out1)

        return carry

    lax.fori_loop(0, GPW, group, 0)

    pltpu.make_async_copy(
        out_v.at[pl.ds(0, OUW)],
        out_hbm.at[pl.ds((gbase + GPW - 2) * OUW, OUW)], s_out0).wait()
    pltpu.make_async_copy(
        out_v.at[pl.ds(OUW, OUW)],
        out_hbm.at[pl.ds((gbase + GPW - 1) * OUW, OUW)], s_out1).wait()
    wait_in(GPW - 1, 0, s_in0)


@functools.partial(jax.jit)
def kernel(spacing_bins, weights, nears, fars):
    nf = jnp.concatenate([nears, fars], axis=1)

    mesh = plsc.VectorSubcoreMesh(core_axis_name="c", subcore_axis_name="s")
    k = pl.kernel(
        _body,
        out_type=jax.ShapeDtypeStruct((R * OUTW,), _f32),
        mesh=mesh,
        compiler_params=pltpu.CompilerParams(needs_layout_passes=False),
        scratch_types=[
            pltpu.VMEM((2 * SPW,), _f32),
            pltpu.VMEM((2 * WTW,), _f32),
            pltpu.VMEM((2 * NFW,), _f32),
            pltpu.VMEM((2 * OUW,), _f32),
            pltpu.VMEM((NB * L,), _f32),
            pltpu.VMEM(((NB + 1) * L,), _i32),
            pltpu.VMEM((NB * L,), _i32),
            pltpu.SemaphoreType.DMA,
            pltpu.SemaphoreType.DMA,
            pltpu.SemaphoreType.DMA,
            pltpu.SemaphoreType.DMA,
        ],
    )
    out = k(spacing_bins.reshape(-1), weights.reshape(-1),
            nf.reshape(-1))
    return out.reshape(R, OUTW)
